# baseline (speedup 1.0000x reference)
import functools

import jax
import jax.numpy as jnp
from jax import lax
from jax.experimental import pallas as pl
from jax.experimental.pallas import tpu as pltpu
from jax.experimental.pallas import tpu_sc as plsc

B_TOK, SEQ = 16384, 50
V, D = 1000000, 64
B = B_TOK * SEQ
NC, NS = 2, 16
NW = NC * NS
B_PER_W = B // NW
CHUNK = 640
SUB = 128
K = CHUNK // SUB
N_CHUNKS = B_PER_W // CHUNK
NBUF = 2
N_GROUPS = N_CHUNKS // NBUF

_mesh = plsc.VectorSubcoreMesh(core_axis_name="c", subcore_axis_name="s")


@functools.partial(
    pl.kernel,
    mesh=_mesh,
    out_type=jax.ShapeDtypeStruct((B, D), jnp.bfloat16),
    scratch_types=[
        pltpu.VMEM((NBUF, CHUNK), jnp.int32),
        pltpu.VMEM((NBUF, CHUNK, D), jnp.bfloat16),
        pltpu.SemaphoreType.DMA((NBUF,)),
        pltpu.SemaphoreType.DMA((NBUF,)),
    ],
    compiler_params=pltpu.CompilerParams(use_tc_tiling_on_sc=False),
)
def _gather_kernel(table_hbm, idx_hbm, out_hbm, idx_v, rows_v, gsem, wsem):
    wid = lax.axis_index("s") * NC + lax.axis_index("c")
    base = wid * B_PER_W

    def fire_gathers(b):
        for j in range(K):
            pltpu.async_copy(
                table_hbm.at[idx_v.at[b, pl.ds(j * SUB, SUB)]],
                rows_v.at[b, pl.ds(j * SUB, SUB)], gsem.at[b])

    def start_gather(ck, b):
        off = base + ck * CHUNK
        pltpu.sync_copy(idx_hbm.at[pl.ds(off, CHUNK)], idx_v.at[b])
        fire_gathers(b)

    for b in range(NBUF):
        start_gather(b, b)

    def body(i, carry):
        for b in range(NBUF):
            ck = i * NBUF + b
            pltpu.make_async_copy(table_hbm.at[idx_v.at[b]], rows_v.at[b],
                                  gsem.at[b]).wait()
            pltpu.async_copy(
                rows_v.at[b], out_hbm.at[pl.ds(base + ck * CHUNK, CHUNK)],
                wsem.at[b])
            nk = ck + NBUF
            off = base + nk * CHUNK
            pltpu.sync_copy(idx_hbm.at[pl.ds(off, CHUNK)], idx_v.at[b])
            pltpu.make_async_copy(
                rows_v.at[b], out_hbm.at[pl.ds(base + ck * CHUNK, CHUNK)],
                wsem.at[b]).wait()
            fire_gathers(b)
        return carry

    lax.fori_loop(0, N_GROUPS - 1, body, 0)

    for b in range(NBUF):
        ck = (N_GROUPS - 1) * NBUF + b
        pltpu.make_async_copy(table_hbm.at[idx_v.at[b]], rows_v.at[b],
                              gsem.at[b]).wait()
        pltpu.async_copy(rows_v.at[b],
                         out_hbm.at[pl.ds(base + ck * CHUNK, CHUNK)],
                         wsem.at[b])
    for b in range(NBUF):
        ck = (N_GROUPS - 1) * NBUF + b
        pltpu.make_async_copy(rows_v.at[b],
                              out_hbm.at[pl.ds(base + ck * CHUNK, CHUNK)],
                              wsem.at[b]).wait()


def kernel(token_ids, weights):
    flat = jnp.swapaxes(token_ids, 0, 1).reshape(-1).astype(jnp.int32)
    out = _gather_kernel(weights.astype(jnp.bfloat16), flat)
    return jnp.swapaxes(out.reshape(SEQ, B_TOK, D), 0, 1).astype(jnp.float32)


# speedup vs baseline: 1.4621x; 1.4621x over previous
import functools

import jax
import jax.numpy as jnp
from jax import lax
from jax.experimental import pallas as pl
from jax.experimental.pallas import tpu as pltpu
from jax.experimental.pallas import tpu_sc as plsc

B_TOK, SEQ = 16384, 50
V, D = 1000000, 64
B = B_TOK * SEQ
NC, NS = 2, 16
NW = NC * NS
B_PER_W = B // NW
CHUNK = 640
SUB = 128
K = CHUNK // SUB
N_CHUNKS = B_PER_W // CHUNK
NBUF = 2
N_GROUPS = N_CHUNKS // NBUF

_mesh = plsc.VectorSubcoreMesh(core_axis_name="c", subcore_axis_name="s")


@functools.partial(
    pl.kernel,
    mesh=_mesh,
    out_type=jax.ShapeDtypeStruct((B, D), jnp.float32),
    scratch_types=[
        pltpu.VMEM((NBUF, CHUNK), jnp.int32),
        pltpu.VMEM((NBUF, CHUNK, D), jnp.float32),
        pltpu.SemaphoreType.DMA((NBUF,)),
        pltpu.SemaphoreType.DMA((NBUF,)),
    ],
    compiler_params=pltpu.CompilerParams(use_tc_tiling_on_sc=False),
)
def _gather_kernel(table_hbm, idx_hbm, out_hbm, idx_v, rows_v, gsem, wsem):
    wid = lax.axis_index("s") * NC + lax.axis_index("c")
    base = wid * B_PER_W

    def fire_gathers(b):
        for j in range(K):
            pltpu.async_copy(
                table_hbm.at[idx_v.at[b, pl.ds(j * SUB, SUB)]],
                rows_v.at[b, pl.ds(j * SUB, SUB)], gsem.at[b])

    def start_gather(ck, b):
        off = base + ck * CHUNK
        pltpu.sync_copy(idx_hbm.at[pl.ds(off, CHUNK)], idx_v.at[b])
        fire_gathers(b)

    for b in range(NBUF):
        start_gather(b, b)

    def body(i, carry):
        for b in range(NBUF):
            ck = i * NBUF + b
            pltpu.make_async_copy(table_hbm.at[idx_v.at[b]], rows_v.at[b],
                                  gsem.at[b]).wait()
            pltpu.async_copy(
                rows_v.at[b], out_hbm.at[pl.ds(base + ck * CHUNK, CHUNK)],
                wsem.at[b])
            nk = ck + NBUF
            off = base + nk * CHUNK
            pltpu.sync_copy(idx_hbm.at[pl.ds(off, CHUNK)], idx_v.at[b])
            pltpu.make_async_copy(
                rows_v.at[b], out_hbm.at[pl.ds(base + ck * CHUNK, CHUNK)],
                wsem.at[b]).wait()
            fire_gathers(b)
        return carry

    lax.fori_loop(0, N_GROUPS - 1, body, 0)

    for b in range(NBUF):
        ck = (N_GROUPS - 1) * NBUF + b
        pltpu.make_async_copy(table_hbm.at[idx_v.at[b]], rows_v.at[b],
                              gsem.at[b]).wait()
        pltpu.async_copy(rows_v.at[b],
                         out_hbm.at[pl.ds(base + ck * CHUNK, CHUNK)],
                         wsem.at[b])
    for b in range(NBUF):
        ck = (N_GROUPS - 1) * NBUF + b
        pltpu.make_async_copy(rows_v.at[b],
                              out_hbm.at[pl.ds(base + ck * CHUNK, CHUNK)],
                              wsem.at[b]).wait()


def kernel(token_ids, weights):
    flat = jnp.swapaxes(token_ids, 0, 1).reshape(-1).astype(jnp.int32)
    out = _gather_kernel(weights, flat)
    return jnp.swapaxes(out.reshape(SEQ, B_TOK, D), 0, 1)
